# baseline (speedup 1.0000x reference)
import functools

import jax
import jax.numpy as jnp
from jax.experimental import pallas as pl
from jax.experimental.pallas import tpu as pltpu

B = 64
R = 10
F = 768
P = 500
Z = 300


def _body(bs, txt_ref, vis_ref, usr_ref, rt_ref, rv_ref, ru_ref, sim_ref,
          lab_ref, Wtxt_ref, btxt_ref, Wvis_ref, bvis_ref, Wusr_ref, busr_ref,
          Wrtxt_ref, brtxt_ref, Wrvis_ref, brvis_ref, Wrusr_ref, brusr_ref,
          Wth_ref, bth_ref, Wlbl_ref, blbl_ref,
          Wp1_ref, bp1_ref, Wp2_ref, bp2_ref, Wp3_ref, bp3_ref, out_ref):
    i = pl.program_id(0)
    bf = jnp.bfloat16

    def dot(a, b):
        return jnp.dot(a.astype(bf), b.astype(bf),
                       preferred_element_type=jnp.float32)

    t0 = jnp.tanh(dot(txt_ref[...], Wtxt_ref[...]) + btxt_ref[...])
    t1 = jnp.tanh(dot(vis_ref[...], Wvis_ref[...]) + bvis_ref[...])
    t2 = jnp.tanh(dot(usr_ref[...], Wusr_ref[...]) + busr_ref[...])
    rt = jnp.tanh(dot(rt_ref[...], Wrtxt_ref[...]) + brtxt_ref[...])
    rv = jnp.tanh(dot(rv_ref[...], Wrvis_ref[...]) + brvis_ref[...])
    ru = jnp.tanh(dot(ru_ref[...], Wrusr_ref[...]) + brusr_ref[...])

    S_t = jnp.sum(rt.reshape(B, R, P), axis=1)
    S_v = jnp.sum(rv.reshape(B, R, P), axis=1)
    S_u = jnp.sum(ru.reshape(B, R, P), axis=1)

    c0 = (t0 + t1 + t2) * (1.0 / 3.0)
    c1 = (t0 + S_t) * (1.0 / 11.0)
    c2 = (t1 + S_v) * (1.0 / 11.0)
    c3 = (t2 + S_u) * (1.0 / 11.0)

    s = jax.nn.softmax(sim_ref[...], axis=1)
    agg_lab = jnp.sum(s * lab_ref[...], axis=1, keepdims=True)

    q = 0.5 * dot(s[0:1, :], rt[0:R, :] + rv[0:R, :])
    row0 = (jax.lax.broadcasted_iota(jnp.int32, (B, 1), 0) == 0) & (i == 0)
    scale = 1.0 / (bs + 1.0)
    d3 = jnp.where(row0, (c2 + bs * q) * scale, c2)
    d4 = jnp.where(row0, (c1 + bs * q) * scale, c1)

    D = jnp.concatenate([(c0 + c1) * 0.5, (c0 + c2) * 0.5,
                         (c0 + c3) * 0.5, d3, d4, c3], axis=0)
    O = dot(D, Wth_ref[...]) + bth_ref[...]

    lab_emb = jax.nn.relu(agg_lab * Wlbl_ref[...] + blbl_ref[...])

    h = bp1_ref[...] + dot(lab_emb, Wp1_ref[6])
    for k in range(6):
        h = h + dot(O[k * B:(k + 1) * B, :], Wp1_ref[k])
    h = jax.nn.relu(h)
    h = jax.nn.relu(dot(h, Wp2_ref[...]) + bp2_ref[...])
    out_ref[...] = jax.nn.sigmoid(dot(h, Wp3_ref[...]) + bp3_ref[...])


def kernel(visual_feature, textual_feature, similarity,
           retrieved_visual_feature, retrieved_textual_feature,
           retrieved_label, user, retrieved_user, retrieved_user_similarity,
           W_vis, b_vis, W_txt, b_txt, W_usr, b_usr, W_rvis, b_rvis,
           W_rtxt, b_rtxt, W_rusr, b_rusr, W_theta, b_theta, W_lbl, b_lbl,
           W_p1, b_p1, W_p2, b_p2, W_p3, b_p3):
    bs = visual_feature.shape[0]
    txt2 = textual_feature.reshape(bs, F)
    vis2 = visual_feature.reshape(bs, F)
    rt2 = retrieved_textual_feature.reshape(bs * R, F)
    rv2 = retrieved_visual_feature.reshape(bs * R, F)
    ru2 = retrieved_user.reshape(bs * R, F)
    lab2 = retrieved_label.reshape(bs, R)
    Wp1r = W_p1.reshape(7, Z, 800)

    bm = lambda i: (i, 0)
    cm = lambda i: (0, 0)

    in_specs = [
        pl.BlockSpec((B, F), bm),
        pl.BlockSpec((B, F), bm),
        pl.BlockSpec((B, F), bm),
        pl.BlockSpec((B * R, F), bm),
        pl.BlockSpec((B * R, F), bm),
        pl.BlockSpec((B * R, F), bm),
        pl.BlockSpec((B, R), bm),
        pl.BlockSpec((B, R), bm),
        pl.BlockSpec((F, P), cm), pl.BlockSpec((1, P), cm),
        pl.BlockSpec((F, P), cm), pl.BlockSpec((1, P), cm),
        pl.BlockSpec((F, P), cm), pl.BlockSpec((1, P), cm),
        pl.BlockSpec((F, P), cm), pl.BlockSpec((1, P), cm),
        pl.BlockSpec((F, P), cm), pl.BlockSpec((1, P), cm),
        pl.BlockSpec((F, P), cm), pl.BlockSpec((1, P), cm),
        pl.BlockSpec((P, Z), cm), pl.BlockSpec((1, Z), cm),
        pl.BlockSpec((1, Z), cm), pl.BlockSpec((1, Z), cm),
        pl.BlockSpec((7, Z, 800), lambda i: (0, 0, 0)),
        pl.BlockSpec((1, 800), cm),
        pl.BlockSpec((800, 200), cm), pl.BlockSpec((1, 200), cm),
        pl.BlockSpec((200, 1), cm), pl.BlockSpec((1, 1), cm),
    ]

    out = pl.pallas_call(
        functools.partial(_body, float(bs)),
        grid=(bs // B,),
        in_specs=in_specs,
        out_specs=pl.BlockSpec((B, 1), bm),
        out_shape=jax.ShapeDtypeStruct((bs, 1), jnp.float32),
        compiler_params=pltpu.CompilerParams(
            dimension_semantics=("arbitrary",)),
    )(txt2, vis2, user, rt2, rv2, ru2, similarity, lab2,
      W_txt, b_txt.reshape(1, P), W_vis, b_vis.reshape(1, P),
      W_usr, b_usr.reshape(1, P), W_rtxt, b_rtxt.reshape(1, P),
      W_rvis, b_rvis.reshape(1, P), W_rusr, b_rusr.reshape(1, P),
      W_theta, b_theta.reshape(1, Z), W_lbl, b_lbl.reshape(1, Z),
      Wp1r, b_p1.reshape(1, 800), W_p2, b_p2.reshape(1, 200),
      W_p3, b_p3.reshape(1, 1))
    return out


# speedup vs baseline: 1.3706x; 1.3706x over previous
import functools

import jax
import jax.numpy as jnp
from jax.experimental import pallas as pl
from jax.experimental.pallas import tpu as pltpu

B = 64
R = 10
F = 768
P = 500
Z = 300


def _body(bs, txt_ref, vis_ref, usr_ref, rt_ref, rv_ref, ru_ref, sim_ref,
          lab_ref, Wtxt_ref, btxt_ref, Wvis_ref, bvis_ref, Wusr_ref, busr_ref,
          Wrtxt_ref, brtxt_ref, Wrvis_ref, brvis_ref, Wrusr_ref, brusr_ref,
          Wth_ref, bth_ref, Wlbl_ref, blbl_ref,
          Wp1_ref, bp1_ref, Wp2_ref, bp2_ref, Wp3_ref, bp3_ref, out_ref):
    i = pl.program_id(0)
    bf = jnp.bfloat16

    def dot(a, b):
        return jnp.dot(a.astype(bf), b.astype(bf),
                       preferred_element_type=jnp.float32)

    t0 = jnp.tanh(dot(txt_ref[...], Wtxt_ref[...]) + btxt_ref[...])
    t1 = jnp.tanh(dot(vis_ref[...], Wvis_ref[...]) + bvis_ref[...])
    t2 = jnp.tanh(dot(usr_ref[...], Wusr_ref[...]) + busr_ref[...])
    rt = jnp.tanh(dot(rt_ref[...].reshape(B * R, F), Wrtxt_ref[...])
                  + brtxt_ref[...])
    rv = jnp.tanh(dot(rv_ref[...].reshape(B * R, F), Wrvis_ref[...])
                  + brvis_ref[...])
    ru = jnp.tanh(dot(ru_ref[...].reshape(B * R, F), Wrusr_ref[...])
                  + brusr_ref[...])

    S_t = jnp.sum(rt.reshape(B, R, P), axis=1)
    S_v = jnp.sum(rv.reshape(B, R, P), axis=1)
    S_u = jnp.sum(ru.reshape(B, R, P), axis=1)

    c0 = (t0 + t1 + t2) * (1.0 / 3.0)
    c1 = (t0 + S_t) * (1.0 / 11.0)
    c2 = (t1 + S_v) * (1.0 / 11.0)
    c3 = (t2 + S_u) * (1.0 / 11.0)

    s = jax.nn.softmax(sim_ref[...], axis=1)
    agg_lab = jnp.sum(s * lab_ref[...], axis=1, keepdims=True)

    q = 0.5 * dot(s[0:1, :], rt[0:R, :] + rv[0:R, :])
    row0 = (jax.lax.broadcasted_iota(jnp.int32, (B, 1), 0) == 0) & (i == 0)
    scale = 1.0 / (bs + 1.0)
    d3 = jnp.where(row0, (c2 + bs * q) * scale, c2)
    d4 = jnp.where(row0, (c1 + bs * q) * scale, c1)

    D = jnp.concatenate([(c0 + c1) * 0.5, (c0 + c2) * 0.5,
                         (c0 + c3) * 0.5, d3, d4, c3], axis=0)
    O = dot(D, Wth_ref[...]) + bth_ref[...]

    lab_emb = jax.nn.relu(agg_lab * Wlbl_ref[...] + blbl_ref[...])

    h = bp1_ref[...] + dot(lab_emb, Wp1_ref[6])
    for k in range(6):
        h = h + dot(O[k * B:(k + 1) * B, :], Wp1_ref[k])
    h = jax.nn.relu(h)
    h = jax.nn.relu(dot(h, Wp2_ref[...]) + bp2_ref[...])
    out_ref[...] = jax.nn.sigmoid(dot(h, Wp3_ref[...]) + bp3_ref[...])


def kernel(visual_feature, textual_feature, similarity,
           retrieved_visual_feature, retrieved_textual_feature,
           retrieved_label, user, retrieved_user, retrieved_user_similarity,
           W_vis, b_vis, W_txt, b_txt, W_usr, b_usr, W_rvis, b_rvis,
           W_rtxt, b_rtxt, W_rusr, b_rusr, W_theta, b_theta, W_lbl, b_lbl,
           W_p1, b_p1, W_p2, b_p2, W_p3, b_p3):
    bs = visual_feature.shape[0]
    txt2 = textual_feature.reshape(bs, F)
    vis2 = visual_feature.reshape(bs, F)
    rt3 = retrieved_textual_feature
    rv3 = retrieved_visual_feature.reshape(bs, R, F)
    ru3 = retrieved_user
    lab2 = retrieved_label.reshape(bs, R)
    Wp1r = W_p1.reshape(7, Z, 800)

    bm = lambda i: (i, 0)
    cm = lambda i: (0, 0)

    in_specs = [
        pl.BlockSpec((B, F), bm),
        pl.BlockSpec((B, F), bm),
        pl.BlockSpec((B, F), bm),
        pl.BlockSpec((B, R, F), lambda i: (i, 0, 0)),
        pl.BlockSpec((B, R, F), lambda i: (i, 0, 0)),
        pl.BlockSpec((B, R, F), lambda i: (i, 0, 0)),
        pl.BlockSpec((B, R), bm),
        pl.BlockSpec((B, R), bm),
        pl.BlockSpec((F, P), cm), pl.BlockSpec((1, P), cm),
        pl.BlockSpec((F, P), cm), pl.BlockSpec((1, P), cm),
        pl.BlockSpec((F, P), cm), pl.BlockSpec((1, P), cm),
        pl.BlockSpec((F, P), cm), pl.BlockSpec((1, P), cm),
        pl.BlockSpec((F, P), cm), pl.BlockSpec((1, P), cm),
        pl.BlockSpec((F, P), cm), pl.BlockSpec((1, P), cm),
        pl.BlockSpec((P, Z), cm), pl.BlockSpec((1, Z), cm),
        pl.BlockSpec((1, Z), cm), pl.BlockSpec((1, Z), cm),
        pl.BlockSpec((7, Z, 800), lambda i: (0, 0, 0)),
        pl.BlockSpec((1, 800), cm),
        pl.BlockSpec((800, 200), cm), pl.BlockSpec((1, 200), cm),
        pl.BlockSpec((200, 1), cm), pl.BlockSpec((1, 1), cm),
    ]

    out = pl.pallas_call(
        functools.partial(_body, float(bs)),
        grid=(bs // B,),
        in_specs=in_specs,
        out_specs=pl.BlockSpec((B, 1), bm),
        out_shape=jax.ShapeDtypeStruct((bs, 1), jnp.float32),
        compiler_params=pltpu.CompilerParams(
            dimension_semantics=("arbitrary",)),
    )(txt2, vis2, user, rt3, rv3, ru3, similarity, lab2,
      W_txt, b_txt.reshape(1, P), W_vis, b_vis.reshape(1, P),
      W_usr, b_usr.reshape(1, P), W_rtxt, b_rtxt.reshape(1, P),
      W_rvis, b_rvis.reshape(1, P), W_rusr, b_rusr.reshape(1, P),
      W_theta, b_theta.reshape(1, Z), W_lbl, b_lbl.reshape(1, Z),
      Wp1r, b_p1.reshape(1, 800), W_p2, b_p2.reshape(1, 200),
      W_p3, b_p3.reshape(1, 1))
    return out
